# baseline (speedup 1.0000x reference)
import functools

import jax
import jax.numpy as jnp
from jax import lax
from jax.experimental import pallas as pl
from jax.experimental.pallas import tpu as pltpu
from jax.experimental.pallas import tpu_sc as plsc


def _sc_gather(u2, i2, mf_user_table, mf_item_table, mlp_user_table,
               mlp_item_table, B, CH):
  MF = mf_user_table.shape[1]
  EMB = mlp_user_table.shape[1]
  info = plsc.get_sparse_core_info()
  NC, NS = info.num_cores, info.num_subcores
  NW = NC * NS
  assert B % (NW * CH) == 0
  b_per_w = B // NW
  n_ch = b_per_w // CH

  mesh = plsc.VectorSubcoreMesh(
      core_axis_name="c", subcore_axis_name="s",
      num_cores=NC, num_subcores=NS)

  @functools.partial(
      pl.kernel,
      out_type=(
          jax.ShapeDtypeStruct((B, MF), jnp.float32),
          jax.ShapeDtypeStruct((B, MF), jnp.float32),
          jax.ShapeDtypeStruct((B, EMB), jnp.float32),
          jax.ShapeDtypeStruct((B, EMB), jnp.float32),
      ),
      mesh=mesh,
      compiler_params=pltpu.CompilerParams(use_tc_tiling_on_sc=False),
      scratch_types=[
          pltpu.VMEM((n_ch, CH), jnp.int32),
          pltpu.VMEM((n_ch, CH), jnp.int32),
          pltpu.VMEM((b_per_w, MF), jnp.float32),
          pltpu.VMEM((b_per_w, MF), jnp.float32),
          pltpu.VMEM((b_per_w, EMB), jnp.float32),
          pltpu.VMEM((b_per_w, EMB), jnp.float32),
          pltpu.SemaphoreType.DMA,
      ],
  )
  def gather_k(u2_hbm, i2_hbm, mfu_hbm, mfi_hbm, mpu_hbm, mpi_hbm,
               o_mfu, o_mfi, o_mpu, o_mpi,
               uv, iv, r_mfu, r_mfi, r_mpu, r_mpi, sem):
    wid = lax.axis_index("s") * NC + lax.axis_index("c")
    row0 = wid * n_ch
    pltpu.sync_copy(u2_hbm.at[pl.ds(row0, n_ch)], uv)
    pltpu.sync_copy(i2_hbm.at[pl.ds(row0, n_ch)], iv)
    handles = []
    for j in range(n_ch):
      sl = pl.ds(j * CH, CH)
      handles.append(pltpu.async_copy(mfu_hbm.at[uv.at[j]], r_mfu.at[sl], sem))
      handles.append(pltpu.async_copy(mfi_hbm.at[iv.at[j]], r_mfi.at[sl], sem))
      handles.append(pltpu.async_copy(mpu_hbm.at[uv.at[j]], r_mpu.at[sl], sem))
      handles.append(pltpu.async_copy(mpi_hbm.at[iv.at[j]], r_mpi.at[sl], sem))
    for h in handles:
      h.wait()
    base = pl.ds(wid * b_per_w, b_per_w)
    pltpu.sync_copy(r_mfu, o_mfu.at[base])
    pltpu.sync_copy(r_mfi, o_mfi.at[base])
    pltpu.sync_copy(r_mpu, o_mpu.at[base])
    pltpu.sync_copy(r_mpi, o_mpi.at[base])

  return gather_k(u2, i2, mf_user_table, mf_item_table,
                  mlp_user_table, mlp_item_table)


def _mlp_body(mfu, mfi, mpu, mpi, w1u, w1i, b1r, w2, b2r, wpm, wph, bpr, out):
  h = jnp.dot(mpu[...], w1u[...], preferred_element_type=jnp.float32)
  h += jnp.dot(mpi[...], w1i[...], preferred_element_type=jnp.float32)
  h = jnp.maximum(h + b1r[...], 0.0)
  h = jnp.dot(h, w2[...], preferred_element_type=jnp.float32)
  h = jnp.maximum(h + b2r[...], 0.0)
  mf = mfu[...] * mfi[...]
  z = jnp.dot(mf, wpm[...], preferred_element_type=jnp.float32)
  z += jnp.dot(h, wph[...], preferred_element_type=jnp.float32)
  out[...] = jax.nn.sigmoid(z + bpr[...])


def kernel(user_indices, item_indices, mf_user_table, mf_item_table,
           mlp_user_table, mlp_item_table, W1, b1, W2, b2, Wp, bp):
  B = user_indices.shape[0]
  MF = mf_user_table.shape[1]
  EMB = mlp_user_table.shape[1]
  L1 = W1.shape[1]
  L2 = W2.shape[1]
  CH = 128

  u2 = user_indices.astype(jnp.int32).reshape(B // CH, CH)
  i2 = item_indices.astype(jnp.int32).reshape(B // CH, CH)

  mf_u, mf_i, mlp_u, mlp_i = _sc_gather(
      u2, i2, mf_user_table, mf_item_table, mlp_user_table, mlp_item_table,
      B, CH)

  W1u, W1i = W1[:EMB], W1[EMB:]
  Wp_mf, Wp_h = Wp[:MF], Wp[MF:]
  b1r = b1.reshape(1, L1)
  b2r = b2.reshape(1, L2)
  bpr = bp.reshape(1, 1)

  BL = 2048
  grid = B // BL
  full = lambda i: (0, 0)
  rows = lambda i: (i, 0)
  pred = pl.pallas_call(
      _mlp_body,
      grid=(grid,),
      in_specs=[
          pl.BlockSpec((BL, MF), rows),
          pl.BlockSpec((BL, MF), rows),
          pl.BlockSpec((BL, EMB), rows),
          pl.BlockSpec((BL, EMB), rows),
          pl.BlockSpec((EMB, L1), full),
          pl.BlockSpec((EMB, L1), full),
          pl.BlockSpec((1, L1), full),
          pl.BlockSpec((L1, L2), full),
          pl.BlockSpec((1, L2), full),
          pl.BlockSpec((MF, 1), full),
          pl.BlockSpec((L2, 1), full),
          pl.BlockSpec((1, 1), full),
      ],
      out_specs=pl.BlockSpec((BL, 1), rows),
      out_shape=jax.ShapeDtypeStruct((B, 1), jnp.float32),
  )(mf_u, mf_i, mlp_u, mlp_i, W1u, W1i, b1r, W2, b2r, Wp_mf, Wp_h, bpr)
  return pred[:, 0]


# speedup vs baseline: 3.4023x; 3.4023x over previous
import functools

import jax
import jax.numpy as jnp
from jax import lax
from jax.experimental import pallas as pl
from jax.experimental.pallas import tpu as pltpu
from jax.experimental.pallas import tpu_sc as plsc


def _sc_gather_cols(u2, i2, mfuT, mfiT, mpuT, mpiT, B):
  MF = mfuT.shape[0]
  EMB = mpuT.shape[0]
  info = plsc.get_sparse_core_info()
  NC, NS = info.num_cores, info.num_subcores
  NW = NC * NS
  assert u2.shape == (NW, B // NW)
  bw = B // NW

  mesh = plsc.VectorSubcoreMesh(
      core_axis_name="c", subcore_axis_name="s",
      num_cores=NC, num_subcores=NS)

  @functools.partial(
      pl.kernel,
      out_type=(
          jax.ShapeDtypeStruct((B * MF,), jnp.float32),
          jax.ShapeDtypeStruct((B * MF,), jnp.float32),
          jax.ShapeDtypeStruct((B * EMB,), jnp.float32),
          jax.ShapeDtypeStruct((B * EMB,), jnp.float32),
      ),
      mesh=mesh,
      compiler_params=pltpu.CompilerParams(needs_layout_passes=False),
      scratch_types=[
          pltpu.VMEM((bw,), jnp.int32),
          pltpu.VMEM((bw,), jnp.int32),
          pltpu.VMEM((MF, 128), jnp.float32),
          pltpu.VMEM((MF, 128), jnp.float32),
          pltpu.VMEM((EMB, 128), jnp.float32),
          pltpu.VMEM((EMB, 128), jnp.float32),
          pltpu.VMEM((MF, 128), jnp.float32),
          pltpu.VMEM((MF, 128), jnp.float32),
          pltpu.VMEM((EMB, 128), jnp.float32),
          pltpu.VMEM((EMB, 128), jnp.float32),
          pltpu.VMEM((bw * MF,), jnp.float32),
          pltpu.VMEM((bw * MF,), jnp.float32),
          pltpu.VMEM((bw * EMB,), jnp.float32),
          pltpu.VMEM((bw * EMB,), jnp.float32),
          pltpu.SemaphoreType.DMA,
          pltpu.SemaphoreType.DMA,
      ],
  )
  def gather_k(u2_hbm, i2_hbm, mfu, mfi, mpu, mpi,
               o_mfu, o_mfi, o_mpu, o_mpi,
               uix, iix,
               a_mfu, a_mfi, a_mpu, a_mpi,
               b_mfu, b_mfi, b_mpu, b_mpi,
               r_mfu, r_mfi, r_mpu, r_mpi,
               sem_a, sem_b):
    wid = lax.axis_index("s") * NC + lax.axis_index("c")
    base = wid * bw
    pltpu.sync_copy(u2_hbm.at[wid], uix)
    pltpu.sync_copy(i2_hbm.at[wid], iix)
    lane = lax.iota(jnp.int32, 16)
    bufs_a = (a_mfu, a_mfi, a_mpu, a_mpi)
    bufs_b = (b_mfu, b_mfi, b_mpu, b_mpi)
    tabs = (mfu, mfi, mpu, mpi)

    def calc(b):
      g = pl.multiple_of((b >> 4) * 16, 16)
      vu = uix[pl.ds(g, 16)]
      vi = iix[pl.ds(g, 16)]
      m = lane == (b & 15)
      neg = jnp.full((16,), -1, jnp.int32)
      cu = jnp.max(jnp.where(m, vu, neg))
      ci = jnp.max(jnp.where(m, vi, neg))
      return cu, ci

    def fire(cu, ci, bufs, sem):
      cu_t = pl.multiple_of((cu >> 7) * 128, 128)
      ci_t = pl.multiple_of((ci >> 7) * 128, 128)
      offs = (cu_t, ci_t, cu_t, ci_t)
      for tab, off, buf in zip(tabs, offs, bufs):
        pltpu.async_copy(tab.at[:, pl.ds(off, 128)], buf, sem)

    def drain(bufs, sem):
      for tab, buf in zip(tabs, bufs):
        pltpu.make_async_copy(tab.at[:, pl.ds(0, 128)], buf, sem).wait()

    def extract(cu, ci, bufs, b):
      cm_u = jnp.broadcast_to(cu & 127, (16,))
      cm_i = jnp.broadcast_to(ci & 127, (16,))
      row_u = plsc.load_gather(bufs[0], [lane, cm_u])
      row_i = plsc.load_gather(bufs[1], [lane, cm_i])
      o16 = pl.multiple_of(b * MF, MF)
      r_mfu[pl.ds(o16, 16)] = row_u
      r_mfi[pl.ds(o16, 16)] = row_i
      plo_u = plsc.load_gather(bufs[2], [lane, cm_u])
      phi_u = plsc.load_gather(bufs[2], [lane + 16, cm_u])
      plo_i = plsc.load_gather(bufs[3], [lane, cm_i])
      phi_i = plsc.load_gather(bufs[3], [lane + 16, cm_i])
      o32 = pl.multiple_of(b * EMB, EMB)
      r_mpu[pl.ds(o32, 16)] = plo_u
      r_mpu[pl.ds(o32 + 16, 16)] = phi_u
      r_mpi[pl.ds(o32, 16)] = plo_i
      r_mpi[pl.ds(o32 + 16, 16)] = phi_i

    cu0, ci0 = calc(0)
    fire(cu0, ci0, bufs_a, sem_a)

    def body(b, carry):
      cu_b, ci_b = carry
      bn = jnp.minimum(b + 1, bw - 1)
      cu_n, ci_n = calc(bn)

      @pl.when(b % 2 == 0)
      def _():
        fire(cu_n, ci_n, bufs_b, sem_b)
        drain(bufs_a, sem_a)
        extract(cu_b, ci_b, bufs_a, b)

      @pl.when(b % 2 == 1)
      def _():
        fire(cu_n, ci_n, bufs_a, sem_a)
        drain(bufs_b, sem_b)
        extract(cu_b, ci_b, bufs_b, b)

      return cu_n, ci_n

    lax.fori_loop(0, bw, body, (cu0, ci0))
    drain(bufs_a, sem_a)

    pltpu.sync_copy(r_mfu, o_mfu.at[pl.ds(base * MF, bw * MF)])
    pltpu.sync_copy(r_mfi, o_mfi.at[pl.ds(base * MF, bw * MF)])
    pltpu.sync_copy(r_mpu, o_mpu.at[pl.ds(base * EMB, bw * EMB)])
    pltpu.sync_copy(r_mpi, o_mpi.at[pl.ds(base * EMB, bw * EMB)])

  return gather_k(u2, i2, mfuT, mfiT, mpuT, mpiT)


def _mlp_body(mfu_r, mfi_r, mpu_r, mpi_r, w1u, w1i, b1r, w2, b2r, wpm, wph,
              bpr, out):
  mfu = mfu_r[...]
  mfi = mfi_r[...]
  mpu = mpu_r[...]
  mpi = mpi_r[...]
  h = jnp.dot(mpu, w1u[...], preferred_element_type=jnp.float32)
  h += jnp.dot(mpi, w1i[...], preferred_element_type=jnp.float32)
  h = jnp.maximum(h + b1r[...], 0.0)
  h = jnp.dot(h, w2[...], preferred_element_type=jnp.float32)
  h = jnp.maximum(h + b2r[...], 0.0)
  mf = mfu * mfi
  z = jnp.dot(mf, wpm[...], preferred_element_type=jnp.float32)
  z += jnp.dot(h, wph[...], preferred_element_type=jnp.float32)
  out[...] = jax.nn.sigmoid(z + bpr[...])


def kernel(user_indices, item_indices, mf_user_table, mf_item_table,
           mlp_user_table, mlp_item_table, W1, b1, W2, b2, Wp, bp):
  B = user_indices.shape[0]
  MF = mf_user_table.shape[1]
  EMB = mlp_user_table.shape[1]
  L1 = W1.shape[1]
  L2 = W2.shape[1]
  NW = 32

  u2 = user_indices.astype(jnp.int32).reshape(NW, B // NW)
  i2 = item_indices.astype(jnp.int32).reshape(NW, B // NW)

  f_mfu, f_mfi, f_mpu, f_mpi = _sc_gather_cols(
      u2, i2, mf_user_table.T, mf_item_table.T,
      mlp_user_table.T, mlp_item_table.T, B)

  mfu = f_mfu.reshape(B, MF)
  mfi = f_mfi.reshape(B, MF)
  mpu = f_mpu.reshape(B, EMB)
  mpi = f_mpi.reshape(B, EMB)

  W1u, W1i = W1[:EMB], W1[EMB:]
  Wp_mf, Wp_h = Wp[:MF], Wp[MF:]
  b1r = b1.reshape(1, L1)
  b2r = b2.reshape(1, L2)
  bpr = bp.reshape(1, 1)

  BL = 2048
  grid = B // BL
  full = lambda i: (0, 0)
  rows = lambda i: (i, 0)
  pred = pl.pallas_call(
      _mlp_body,
      grid=(grid,),
      in_specs=[
          pl.BlockSpec((BL, MF), rows),
          pl.BlockSpec((BL, MF), rows),
          pl.BlockSpec((BL, EMB), rows),
          pl.BlockSpec((BL, EMB), rows),
          pl.BlockSpec((EMB, L1), full),
          pl.BlockSpec((EMB, L1), full),
          pl.BlockSpec((1, L1), full),
          pl.BlockSpec((L1, L2), full),
          pl.BlockSpec((1, L2), full),
          pl.BlockSpec((MF, 1), full),
          pl.BlockSpec((L2, 1), full),
          pl.BlockSpec((1, 1), full),
      ],
      out_specs=pl.BlockSpec((BL, 1), rows),
      out_shape=jax.ShapeDtypeStruct((B, 1), jnp.float32),
  )(mfu, mfi, mpu, mpi, W1u, W1i, b1r, W2, b2r, Wp_mf, Wp_h, bpr)
  return pred[:, 0]


# speedup vs baseline: 4.2233x; 1.2413x over previous
import functools

import jax
import jax.numpy as jnp
from jax import lax
from jax.experimental import pallas as pl
from jax.experimental.pallas import tpu as pltpu
from jax.experimental.pallas import tpu_sc as plsc


def _sc_gather_cols(u2, i2, mfuT, mfiT, mpuT, mpiT, B):
  MF = mfuT.shape[0]
  EMB = mpuT.shape[0]
  info = plsc.get_sparse_core_info()
  NC, NS = info.num_cores, info.num_subcores
  NW = NC * NS
  assert u2.shape == (NW, B // NW)
  bw = B // NW

  mesh = plsc.VectorSubcoreMesh(
      core_axis_name="c", subcore_axis_name="s",
      num_cores=NC, num_subcores=NS)

  @functools.partial(
      pl.kernel,
      out_type=(
          jax.ShapeDtypeStruct((B * MF,), jnp.float32),
          jax.ShapeDtypeStruct((B * MF,), jnp.float32),
          jax.ShapeDtypeStruct((B * EMB,), jnp.float32),
          jax.ShapeDtypeStruct((B * EMB,), jnp.float32),
      ),
      mesh=mesh,
      compiler_params=pltpu.CompilerParams(needs_layout_passes=False),
      scratch_types=(
          [pltpu.VMEM((bw,), jnp.int32)] * 2
          + [pltpu.VMEM((MF, 128), jnp.float32),
             pltpu.VMEM((MF, 128), jnp.float32),
             pltpu.VMEM((EMB, 128), jnp.float32),
             pltpu.VMEM((EMB, 128), jnp.float32)] * 4
          + [pltpu.VMEM((bw * MF,), jnp.float32),
             pltpu.VMEM((bw * MF,), jnp.float32),
             pltpu.VMEM((bw * EMB,), jnp.float32),
             pltpu.VMEM((bw * EMB,), jnp.float32)]
          + [pltpu.SemaphoreType.DMA] * 4
      ),
  )
  def gather_k(u2_hbm, i2_hbm, mfu, mfi, mpu, mpi,
               o_mfu, o_mfi, o_mpu, o_mpi, uix, iix, *rest):
    sets = [tuple(rest[4 * k:4 * k + 4]) for k in range(4)]
    r_mfu, r_mfi, r_mpu, r_mpi = rest[16:20]
    sems = rest[20:24]
    wid = lax.axis_index("s") * NC + lax.axis_index("c")
    base = wid * bw
    pltpu.sync_copy(u2_hbm.at[wid], uix)
    pltpu.sync_copy(i2_hbm.at[wid], iix)
    lane = lax.iota(jnp.int32, 16)
    tabs = (mfu, mfi, mpu, mpi)

    def calc(b):
      g = pl.multiple_of((b >> 4) * 16, 16)
      vu = uix[pl.ds(g, 16)]
      vi = iix[pl.ds(g, 16)]
      m = lane == (b & 15)
      neg = jnp.full((16,), -1, jnp.int32)
      cu = jnp.max(jnp.where(m, vu, neg))
      ci = jnp.max(jnp.where(m, vi, neg))
      return cu, ci

    def fire(cu, ci, bufs, sem):
      cu_t = pl.multiple_of((cu >> 7) * 128, 128)
      ci_t = pl.multiple_of((ci >> 7) * 128, 128)
      offs = (cu_t, ci_t, cu_t, ci_t)
      for tab, off, buf in zip(tabs, offs, bufs):
        pltpu.async_copy(tab.at[:, pl.ds(off, 128)], buf, sem)

    def drain(bufs, sem):
      for tab, buf in zip(tabs, bufs):
        pltpu.make_async_copy(tab.at[:, pl.ds(0, 128)], buf, sem).wait()

    def extract(cu, ci, bufs, b):
      cm_u = jnp.broadcast_to(cu & 127, (16,))
      cm_i = jnp.broadcast_to(ci & 127, (16,))
      row_u = plsc.load_gather(bufs[0], [lane, cm_u])
      row_i = plsc.load_gather(bufs[1], [lane, cm_i])
      o16 = pl.multiple_of(b * MF, MF)
      r_mfu[pl.ds(o16, 16)] = row_u
      r_mfi[pl.ds(o16, 16)] = row_i
      plo_u = plsc.load_gather(bufs[2], [lane, cm_u])
      phi_u = plsc.load_gather(bufs[2], [lane + 16, cm_u])
      plo_i = plsc.load_gather(bufs[3], [lane, cm_i])
      phi_i = plsc.load_gather(bufs[3], [lane + 16, cm_i])
      o32 = pl.multiple_of(b * EMB, EMB)
      r_mpu[pl.ds(o32, 16)] = plo_u
      r_mpu[pl.ds(o32 + 16, 16)] = phi_u
      r_mpi[pl.ds(o32, 16)] = plo_i
      r_mpi[pl.ds(o32 + 16, 16)] = phi_i

    D = 4
    carry0 = []
    for b0 in range(D - 1):
      cu_p, ci_p = calc(b0)
      fire(cu_p, ci_p, sets[b0], sems[b0])
      carry0.extend((cu_p, ci_p))

    def body(b, carry):
      cu_b, ci_b = carry[0], carry[1]
      bn = jnp.minimum(b + (D - 1), bw - 1)
      cu_n, ci_n = calc(bn)
      for k in range(D):
        @pl.when(b % D == k)
        def _(k=k):
          fire(cu_n, ci_n, sets[(k + D - 1) % D], sems[(k + D - 1) % D])
          drain(sets[k], sems[k])
          extract(cu_b, ci_b, sets[k], b)
      return (*carry[2:], cu_n, ci_n)

    lax.fori_loop(0, bw, body, tuple(carry0))
    for j in range(D - 1):
      k = (bw + j) % D
      drain(sets[k], sems[k])

    pltpu.sync_copy(r_mfu, o_mfu.at[pl.ds(base * MF, bw * MF)])
    pltpu.sync_copy(r_mfi, o_mfi.at[pl.ds(base * MF, bw * MF)])
    pltpu.sync_copy(r_mpu, o_mpu.at[pl.ds(base * EMB, bw * EMB)])
    pltpu.sync_copy(r_mpi, o_mpi.at[pl.ds(base * EMB, bw * EMB)])

  return gather_k(u2, i2, mfuT, mfiT, mpuT, mpiT)


def _mlp_body(mfu_r, mfi_r, mpu_r, mpi_r, w1u, w1i, b1r, w2, b2r, wpm, wph,
              bpr, out):
  mfu = mfu_r[...]
  mfi = mfi_r[...]
  mpu = mpu_r[...]
  mpi = mpi_r[...]
  h = jnp.dot(mpu, w1u[...], preferred_element_type=jnp.float32)
  h += jnp.dot(mpi, w1i[...], preferred_element_type=jnp.float32)
  h = jnp.maximum(h + b1r[...], 0.0)
  h = jnp.dot(h, w2[...], preferred_element_type=jnp.float32)
  h = jnp.maximum(h + b2r[...], 0.0)
  mf = mfu * mfi
  z = jnp.dot(mf, wpm[...], preferred_element_type=jnp.float32)
  z += jnp.dot(h, wph[...], preferred_element_type=jnp.float32)
  out[...] = jax.nn.sigmoid(z + bpr[...])


def kernel(user_indices, item_indices, mf_user_table, mf_item_table,
           mlp_user_table, mlp_item_table, W1, b1, W2, b2, Wp, bp):
  B = user_indices.shape[0]
  MF = mf_user_table.shape[1]
  EMB = mlp_user_table.shape[1]
  L1 = W1.shape[1]
  L2 = W2.shape[1]
  NW = 32

  u2 = user_indices.astype(jnp.int32).reshape(NW, B // NW)
  i2 = item_indices.astype(jnp.int32).reshape(NW, B // NW)

  f_mfu, f_mfi, f_mpu, f_mpi = _sc_gather_cols(
      u2, i2, mf_user_table.T, mf_item_table.T,
      mlp_user_table.T, mlp_item_table.T, B)

  mfu = f_mfu.reshape(B, MF)
  mfi = f_mfi.reshape(B, MF)
  mpu = f_mpu.reshape(B, EMB)
  mpi = f_mpi.reshape(B, EMB)

  W1u, W1i = W1[:EMB], W1[EMB:]
  Wp_mf, Wp_h = Wp[:MF], Wp[MF:]
  b1r = b1.reshape(1, L1)
  b2r = b2.reshape(1, L2)
  bpr = bp.reshape(1, 1)

  BL = 2048
  grid = B // BL
  full = lambda i: (0, 0)
  rows = lambda i: (i, 0)
  pred = pl.pallas_call(
      _mlp_body,
      grid=(grid,),
      in_specs=[
          pl.BlockSpec((BL, MF), rows),
          pl.BlockSpec((BL, MF), rows),
          pl.BlockSpec((BL, EMB), rows),
          pl.BlockSpec((BL, EMB), rows),
          pl.BlockSpec((EMB, L1), full),
          pl.BlockSpec((EMB, L1), full),
          pl.BlockSpec((1, L1), full),
          pl.BlockSpec((L1, L2), full),
          pl.BlockSpec((1, L2), full),
          pl.BlockSpec((MF, 1), full),
          pl.BlockSpec((L2, 1), full),
          pl.BlockSpec((1, 1), full),
      ],
      out_specs=pl.BlockSpec((BL, 1), rows),
      out_shape=jax.ShapeDtypeStruct((B, 1), jnp.float32),
  )(mfu, mfi, mpu, mpi, W1u, W1i, b1r, W2, b2r, Wp_mf, Wp_h, bpr)
  return pred[:, 0]


# speedup vs baseline: 4.5532x; 1.0781x over previous
import functools

import jax
import jax.numpy as jnp
from jax import lax
from jax.experimental import pallas as pl
from jax.experimental.pallas import tpu as pltpu
from jax.experimental.pallas import tpu_sc as plsc


def _sc_gather_cols(u2, i2, mfuT, mfiT, mpuT, mpiT, B):
  MF = mfuT.shape[0]
  EMB = mpuT.shape[0]
  info = plsc.get_sparse_core_info()
  NC, NS = info.num_cores, info.num_subcores
  NW = NC * NS
  assert u2.shape == (NW, B // NW)
  bw = B // NW

  mesh = plsc.VectorSubcoreMesh(
      core_axis_name="c", subcore_axis_name="s",
      num_cores=NC, num_subcores=NS)

  @functools.partial(
      pl.kernel,
      out_type=(
          jax.ShapeDtypeStruct((B * MF,), jnp.float32),
          jax.ShapeDtypeStruct((B * MF,), jnp.float32),
          jax.ShapeDtypeStruct((B * EMB,), jnp.float32),
          jax.ShapeDtypeStruct((B * EMB,), jnp.float32),
      ),
      mesh=mesh,
      compiler_params=pltpu.CompilerParams(needs_layout_passes=False),
      scratch_types=(
          [pltpu.VMEM((bw,), jnp.int32)] * 2
          + [pltpu.VMEM((MF, 128), jnp.float32),
             pltpu.VMEM((MF, 128), jnp.float32),
             pltpu.VMEM((EMB, 128), jnp.float32),
             pltpu.VMEM((EMB, 128), jnp.float32)] * 6
          + [pltpu.VMEM((bw * MF,), jnp.float32),
             pltpu.VMEM((bw * MF,), jnp.float32),
             pltpu.VMEM((bw * EMB,), jnp.float32),
             pltpu.VMEM((bw * EMB,), jnp.float32)]
          + [pltpu.SemaphoreType.DMA] * 6
      ),
  )
  def gather_k(u2_hbm, i2_hbm, mfu, mfi, mpu, mpi,
               o_mfu, o_mfi, o_mpu, o_mpi, uix, iix, *rest):
    sets = [tuple(rest[4 * k:4 * k + 4]) for k in range(6)]
    r_mfu, r_mfi, r_mpu, r_mpi = rest[24:28]
    sems = rest[28:34]
    wid = lax.axis_index("s") * NC + lax.axis_index("c")
    base = wid * bw
    pltpu.sync_copy(u2_hbm.at[wid], uix)
    pltpu.sync_copy(i2_hbm.at[wid], iix)
    lane = lax.iota(jnp.int32, 16)
    tabs = (mfu, mfi, mpu, mpi)

    def calc(b):
      g = pl.multiple_of((b >> 4) * 16, 16)
      vu = uix[pl.ds(g, 16)]
      vi = iix[pl.ds(g, 16)]
      m = lane == (b & 15)
      neg = jnp.full((16,), -1, jnp.int32)
      cu = jnp.max(jnp.where(m, vu, neg))
      ci = jnp.max(jnp.where(m, vi, neg))
      return cu, ci

    def fire(cu, ci, bufs, sem):
      cu_t = pl.multiple_of((cu >> 7) * 128, 128)
      ci_t = pl.multiple_of((ci >> 7) * 128, 128)
      offs = (cu_t, ci_t, cu_t, ci_t)
      for tab, off, buf in zip(tabs, offs, bufs):
        pltpu.async_copy(tab.at[:, pl.ds(off, 128)], buf, sem)

    def drain(bufs, sem):
      for tab, buf in zip(tabs, bufs):
        pltpu.make_async_copy(tab.at[:, pl.ds(0, 128)], buf, sem).wait()

    def extract(cu, ci, bufs, b):
      cm_u = jnp.broadcast_to(cu & 127, (16,))
      cm_i = jnp.broadcast_to(ci & 127, (16,))
      row_u = plsc.load_gather(bufs[0], [lane, cm_u])
      row_i = plsc.load_gather(bufs[1], [lane, cm_i])
      o16 = pl.multiple_of(b * MF, MF)
      r_mfu[pl.ds(o16, 16)] = row_u
      r_mfi[pl.ds(o16, 16)] = row_i
      plo_u = plsc.load_gather(bufs[2], [lane, cm_u])
      phi_u = plsc.load_gather(bufs[2], [lane + 16, cm_u])
      plo_i = plsc.load_gather(bufs[3], [lane, cm_i])
      phi_i = plsc.load_gather(bufs[3], [lane + 16, cm_i])
      o32 = pl.multiple_of(b * EMB, EMB)
      r_mpu[pl.ds(o32, 16)] = plo_u
      r_mpu[pl.ds(o32 + 16, 16)] = phi_u
      r_mpi[pl.ds(o32, 16)] = plo_i
      r_mpi[pl.ds(o32 + 16, 16)] = phi_i

    D = 6
    carry0 = []
    for b0 in range(D - 1):
      cu_p, ci_p = calc(b0)
      fire(cu_p, ci_p, sets[b0], sems[b0])
      carry0.extend((cu_p, ci_p))

    def body(b, carry):
      cu_b, ci_b = carry[0], carry[1]
      bn = jnp.minimum(b + (D - 1), bw - 1)
      cu_n, ci_n = calc(bn)
      for k in range(D):
        @pl.when(b % D == k)
        def _(k=k):
          fire(cu_n, ci_n, sets[(k + D - 1) % D], sems[(k + D - 1) % D])
          drain(sets[k], sems[k])
          extract(cu_b, ci_b, sets[k], b)
      return (*carry[2:], cu_n, ci_n)

    lax.fori_loop(0, bw, body, tuple(carry0))
    for j in range(D - 1):
      k = (bw + j) % D
      drain(sets[k], sems[k])

    pltpu.sync_copy(r_mfu, o_mfu.at[pl.ds(base * MF, bw * MF)])
    pltpu.sync_copy(r_mfi, o_mfi.at[pl.ds(base * MF, bw * MF)])
    pltpu.sync_copy(r_mpu, o_mpu.at[pl.ds(base * EMB, bw * EMB)])
    pltpu.sync_copy(r_mpi, o_mpi.at[pl.ds(base * EMB, bw * EMB)])

  return gather_k(u2, i2, mfuT, mfiT, mpuT, mpiT)


def _mlp_body(mfu_r, mfi_r, mpu_r, mpi_r, w1u, w1i, b1r, w2, b2r, wpm, wph,
              bpr, out):
  mfu = mfu_r[...]
  mfi = mfi_r[...]
  mpu = mpu_r[...]
  mpi = mpi_r[...]
  h = jnp.dot(mpu, w1u[...], preferred_element_type=jnp.float32)
  h += jnp.dot(mpi, w1i[...], preferred_element_type=jnp.float32)
  h = jnp.maximum(h + b1r[...], 0.0)
  h = jnp.dot(h, w2[...], preferred_element_type=jnp.float32)
  h = jnp.maximum(h + b2r[...], 0.0)
  mf = mfu * mfi
  z = jnp.dot(mf, wpm[...], preferred_element_type=jnp.float32)
  z += jnp.dot(h, wph[...], preferred_element_type=jnp.float32)
  out[...] = jax.nn.sigmoid(z + bpr[...])


def kernel(user_indices, item_indices, mf_user_table, mf_item_table,
           mlp_user_table, mlp_item_table, W1, b1, W2, b2, Wp, bp):
  B = user_indices.shape[0]
  MF = mf_user_table.shape[1]
  EMB = mlp_user_table.shape[1]
  L1 = W1.shape[1]
  L2 = W2.shape[1]
  NW = 32

  u2 = user_indices.astype(jnp.int32).reshape(NW, B // NW)
  i2 = item_indices.astype(jnp.int32).reshape(NW, B // NW)

  f_mfu, f_mfi, f_mpu, f_mpi = _sc_gather_cols(
      u2, i2, mf_user_table.T, mf_item_table.T,
      mlp_user_table.T, mlp_item_table.T, B)

  mfu = f_mfu.reshape(B, MF)
  mfi = f_mfi.reshape(B, MF)
  mpu = f_mpu.reshape(B, EMB)
  mpi = f_mpi.reshape(B, EMB)

  W1u, W1i = W1[:EMB], W1[EMB:]
  Wp_mf, Wp_h = Wp[:MF], Wp[MF:]
  b1r = b1.reshape(1, L1)
  b2r = b2.reshape(1, L2)
  bpr = bp.reshape(1, 1)

  BL = 2048
  grid = B // BL
  full = lambda i: (0, 0)
  rows = lambda i: (i, 0)
  pred = pl.pallas_call(
      _mlp_body,
      grid=(grid,),
      in_specs=[
          pl.BlockSpec((BL, MF), rows),
          pl.BlockSpec((BL, MF), rows),
          pl.BlockSpec((BL, EMB), rows),
          pl.BlockSpec((BL, EMB), rows),
          pl.BlockSpec((EMB, L1), full),
          pl.BlockSpec((EMB, L1), full),
          pl.BlockSpec((1, L1), full),
          pl.BlockSpec((L1, L2), full),
          pl.BlockSpec((1, L2), full),
          pl.BlockSpec((MF, 1), full),
          pl.BlockSpec((L2, 1), full),
          pl.BlockSpec((1, 1), full),
      ],
      out_specs=pl.BlockSpec((BL, 1), rows),
      out_shape=jax.ShapeDtypeStruct((B, 1), jnp.float32),
  )(mfu, mfi, mpu, mpi, W1u, W1i, b1r, W2, b2r, Wp_mf, Wp_h, bpr)
  return pred[:, 0]
